# Initial kernel scaffold; baseline (speedup 1.0000x reference)
import jax
import jax.numpy as jnp
from jax.experimental import pallas as pl


def kernel(x, edge_index, edge_attr, params):
    raise NotImplementedError("write your pallas kernel here")


# speedup vs baseline: 1.0016x; 1.0016x over previous
import jax
import jax.numpy as jnp
from jax.experimental import pallas as pl


def _lin(p, x):
    return x @ p['w'] + p['b']


def _bn(p, x):
    m = x.mean(0)
    v = x.var(0)
    return (x - m) * jax.lax.rsqrt(v + 1e-5) * p['g'] + p['b']


def _hardsig(x):
    return jnp.clip((x + 3.0) / 6.0, 0.0, 1.0)


def _nn_edge(p, feat):
    x = jax.nn.relu(_bn(p['bn1'], _lin(p['nn1'], feat)))
    x1 = x
    x = jax.nn.relu(_bn(p['bn2'], _lin(p['nn2'], x)))
    x2 = x
    x = p['c1'][0] * x1 + p['c1'][1] * x2
    x = jax.nn.relu(_bn(p['bn3'], _lin(p['nn3'], x)))
    x = p['c2'][0] * x1 + p['c2'][1] * x2 + p['c2'][2] * x
    a = p['att']
    f = jax.nn.relu(_bn(a['bn'], _lin(a['l1'], feat)))
    f = _hardsig(f @ a['l2w'])
    return x * f


def _edge_nn(p, ea):
    h = _bn(p['bn_in'], ea)
    h = _nn_edge(p['ne'], h)
    h = _lin(p['lin'], h)
    return _bn(p['bn_out'], h)


def _nnconv(p, x, src, dst, ea, n):
    w = _edge_nn(p['edge_nn'], ea).reshape(-1, 5, 5)
    msg = jnp.einsum('ei,eio->eo', x[src], w)
    aggr = jax.ops.segment_sum(msg, dst, num_segments=n)
    return aggr + x @ p['root'] + p['bias']


def _block(p, x, src, dst, ea, n):
    h = jnp.clip(_bn(p['bn1'], _nnconv(p['convs'][0], x, src, dst, ea, n)), -1.0, 10.0)
    x1 = h
    h = jnp.clip(_bn(p['bn2'], _nnconv(p['convs'][1], h, src, dst, ea, n)), -1.0, 10.0)
    h = jnp.clip(_bn(p['bn3'], _nnconv(p['convs'][2], h, src, dst, ea, n)), -1.0, 10.0)
    return h + x1


def _cross(p, x, feat):
    f = jax.nn.relu(_bn(p['bn'], _lin(p['l1'], feat)))
    f = jnp.clip(f @ p['l2w'], 0.0, 0.9)
    return x * f


def _final_lin_kernel(h_ref, w_ref, b_ref, o_ref):
    o_ref[...] = h_ref[...] @ w_ref[...] + b_ref[0, 0]


def kernel(x, edge_index, edge_attr, params):
    src = edge_index[0]
    dst = edge_index[1]
    n = x.shape[0]
    x = _bn(params['bn0'], x)
    B = params['blocks']
    h1 = _block(B[0], x, src, dst, edge_attr, n); h2 = h1
    h1 = _block(B[1], h1, src, dst, edge_attr, n); x1 = h1
    h1 = _block(B[2], h1, src, dst, edge_attr, n); y1 = h1
    h1 = h1 + h2 + x
    h1 = _block(B[3], h1, src, dst, edge_attr, n); h3 = h1
    h1 = _block(B[4], h1, src, dst, edge_attr, n); x2 = h1
    h1 = _block(B[5], h1, src, dst, edge_attr, n); y2 = h1
    h1 = h1 + h2 + h3
    h1 = _block(B[6], h1, src, dst, edge_attr, n); h4 = h1
    h1 = _block(B[7], h1, src, dst, edge_attr, n); x3 = h1
    h1 = _block(B[8], h1, src, dst, edge_attr, n); y3 = h1
    h1 = h1 + h3 + h4
    h1 = _block(B[9], h1, src, dst, edge_attr, n); h5 = h1
    h1 = _block(B[10], h1, src, dst, edge_attr, n); x4 = h1
    h1 = _block(B[11], h1, src, dst, edge_attr, n); y4 = h1
    h1 = h1 + h4 + h5
    h1 = _block(B[12], h1, src, dst, edge_attr, n); h6 = h1
    h1 = _block(B[13], h1, src, dst, edge_attr, n); x5 = h1
    h1 = _block(B[14], h1, src, dst, edge_attr, n); y5 = h1
    h1 = h1 + h5 + h6
    h1 = _block(B[15], h1, src, dst, edge_attr, n); h7 = h1
    h1 = _block(B[16], h1, src, dst, edge_attr, n); x6 = h1
    h1 = _block(B[17], h1, src, dst, edge_attr, n)
    h = jnp.concatenate((h1, h2, h3, h4, h5, h6, h7, x1, x2, x3, x4, x5, x6, y1, y2, y3, y4, y5), axis=1)
    h = _cross(params['att1'], h, h) + _cross(params['att2'], h, h)
    lw = params['lin_out']['w']
    lb = params['lin_out']['b']
    out = pl.pallas_call(
        _final_lin_kernel,
        out_shape=jax.ShapeDtypeStruct((h.shape[0], 1), jnp.float32),
    )(h, lw, lb.reshape(1, 1))
    return out.squeeze(-1)


# speedup vs baseline: 4.2008x; 4.1943x over previous
import functools

import jax
import jax.numpy as jnp
from jax import lax
from jax.experimental import pallas as pl
from jax.experimental.pallas import tpu as pltpu
from jax.experimental.pallas import tpu_sc as plsc

N_NODES = 10000
N_EDGES = 320000
NC, NS = 2, 16
NW = NC * NS
EPW = N_EDGES // NW
CH = 80
NCH = EPW // CH
NPS = N_NODES // NS


def _mesh():
    return plsc.VectorSubcoreMesh(
        core_axis_name="c", subcore_axis_name="s",
        num_cores=NC, num_subcores=NS)


@functools.partial(
    pl.kernel,
    out_type=jax.ShapeDtypeStruct((N_EDGES, 8), jnp.float32),
    mesh=_mesh(),
    compiler_params=pltpu.CompilerParams(use_tc_tiling_on_sc=False),
    scratch_types=[
        pltpu.VMEM((NCH, CH), jnp.int32),
        pltpu.VMEM((EPW, 8), jnp.float32),
        pltpu.SemaphoreType.DMA,
    ],
)
def _sc_gather_k(x_hbm, src_hbm, out_hbm, idx_v, rows_v, sem):
    wid = lax.axis_index("s") * NC + lax.axis_index("c")
    pltpu.sync_copy(src_hbm.at[wid], idx_v)

    def body(j, carry):
        pltpu.async_copy(
            x_hbm.at[idx_v.at[j]], rows_v.at[pl.ds(j * CH, CH)], sem
        ).wait()
        return carry

    lax.fori_loop(0, NCH, body, 0)
    pltpu.sync_copy(rows_v, out_hbm.at[pl.ds(wid * EPW, EPW)])


@functools.partial(
    pl.kernel,
    out_type=jax.ShapeDtypeStruct((NC, N_NODES, 8), jnp.float32),
    mesh=_mesh(),
    compiler_params=pltpu.CompilerParams(use_tc_tiling_on_sc=False),
    scratch_types=[
        pltpu.VMEM((NCH, CH), jnp.int32),
        pltpu.VMEM((EPW, 8), jnp.float32),
        pltpu.VMEM_SHARED((N_NODES, 8), jnp.float32),
        pltpu.SemaphoreType.DMA,
    ],
)
def _sc_scatter_k(msg_hbm, dst_hbm, zero_hbm, out_hbm, idx_v, msg_v, aggr_sh, sem):
    cid = lax.axis_index("c")
    sid = lax.axis_index("s")
    wid = sid * NC + cid
    pltpu.sync_copy(zero_hbm.at[pl.ds(sid * NPS, NPS)],
                    aggr_sh.at[pl.ds(sid * NPS, NPS)])
    plsc.subcore_barrier()
    pltpu.sync_copy(dst_hbm.at[wid], idx_v)
    pltpu.sync_copy(msg_hbm.at[pl.ds(wid * EPW, EPW)], msg_v)

    def body(j, carry):
        pltpu.sync_copy(msg_v.at[pl.ds(j * CH, CH)],
                        aggr_sh.at[idx_v.at[j]], add=True)
        return carry

    lax.fori_loop(0, NCH, body, 0)
    plsc.subcore_barrier()
    pltpu.sync_copy(aggr_sh.at[pl.ds(sid * NPS, NPS)],
                    out_hbm.at[cid, pl.ds(sid * NPS, NPS)])


def _lin(p, x):
    return x @ p['w'] + p['b']


def _bn(p, x):
    m = x.mean(0)
    v = x.var(0)
    return (x - m) * jax.lax.rsqrt(v + 1e-5) * p['g'] + p['b']


def _hardsig(x):
    return jnp.clip((x + 3.0) / 6.0, 0.0, 1.0)


def _nn_edge(p, feat):
    x = jax.nn.relu(_bn(p['bn1'], _lin(p['nn1'], feat)))
    x1 = x
    x = jax.nn.relu(_bn(p['bn2'], _lin(p['nn2'], x)))
    x2 = x
    x = p['c1'][0] * x1 + p['c1'][1] * x2
    x = jax.nn.relu(_bn(p['bn3'], _lin(p['nn3'], x)))
    x = p['c2'][0] * x1 + p['c2'][1] * x2 + p['c2'][2] * x
    a = p['att']
    f = jax.nn.relu(_bn(a['bn'], _lin(a['l1'], feat)))
    f = _hardsig(f @ a['l2w'])
    return x * f


def _edge_nn(p, ea):
    h = _bn(p['bn_in'], ea)
    h = _nn_edge(p['ne'], h)
    h = _lin(p['lin'], h)
    return _bn(p['bn_out'], h)


def _nnconv(p, x, srcr, dstr, ea, zeros8):
    w = _edge_nn(p['edge_nn'], ea).reshape(-1, 5, 5)
    xp = jnp.pad(x, ((0, 0), (0, 3)))
    xs = _sc_gather_k(xp, srcr)[:, :5]
    msg = jnp.einsum('ei,eio->eo', xs, w)
    msg8 = jnp.pad(msg, ((0, 0), (0, 3)))
    parts = _sc_scatter_k(msg8, dstr, zeros8)
    aggr = (parts[0] + parts[1])[:, :5]
    return aggr + x @ p['root'] + p['bias']


def _block(p, x, srcr, dstr, ea, zeros8):
    h = jnp.clip(_bn(p['bn1'], _nnconv(p['convs'][0], x, srcr, dstr, ea, zeros8)), -1.0, 10.0)
    x1 = h
    h = jnp.clip(_bn(p['bn2'], _nnconv(p['convs'][1], h, srcr, dstr, ea, zeros8)), -1.0, 10.0)
    h = jnp.clip(_bn(p['bn3'], _nnconv(p['convs'][2], h, srcr, dstr, ea, zeros8)), -1.0, 10.0)
    return h + x1


def _cross(p, x, feat):
    f = jax.nn.relu(_bn(p['bn'], _lin(p['l1'], feat)))
    f = jnp.clip(f @ p['l2w'], 0.0, 0.9)
    return x * f


def kernel(x, edge_index, edge_attr, params):
    srcr = edge_index[0].reshape(NW, NCH, CH)
    dstr = edge_index[1].reshape(NW, NCH, CH)
    zeros8 = jnp.zeros((N_NODES, 8), jnp.float32)
    x = _bn(params['bn0'], x)
    B = params['blocks']
    args = (srcr, dstr, edge_attr, zeros8)
    h1 = _block(B[0], x, *args); h2 = h1
    h1 = _block(B[1], h1, *args); x1 = h1
    h1 = _block(B[2], h1, *args); y1 = h1
    h1 = h1 + h2 + x
    h1 = _block(B[3], h1, *args); h3 = h1
    h1 = _block(B[4], h1, *args); x2 = h1
    h1 = _block(B[5], h1, *args); y2 = h1
    h1 = h1 + h2 + h3
    h1 = _block(B[6], h1, *args); h4 = h1
    h1 = _block(B[7], h1, *args); x3 = h1
    h1 = _block(B[8], h1, *args); y3 = h1
    h1 = h1 + h3 + h4
    h1 = _block(B[9], h1, *args); h5 = h1
    h1 = _block(B[10], h1, *args); x4 = h1
    h1 = _block(B[11], h1, *args); y4 = h1
    h1 = h1 + h4 + h5
    h1 = _block(B[12], h1, *args); h6 = h1
    h1 = _block(B[13], h1, *args); x5 = h1
    h1 = _block(B[14], h1, *args); y5 = h1
    h1 = h1 + h5 + h6
    h1 = _block(B[15], h1, *args); h7 = h1
    h1 = _block(B[16], h1, *args); x6 = h1
    h1 = _block(B[17], h1, *args)
    h = jnp.concatenate((h1, h2, h3, h4, h5, h6, h7, x1, x2, x3, x4, x5, x6, y1, y2, y3, y4, y5), axis=1)
    h = _cross(params['att1'], h, h) + _cross(params['att2'], h, h)
    h = _lin(params['lin_out'], h)
    return h.squeeze(-1)
